# baseline (speedup 1.0000x reference)
import functools

import jax
import jax.numpy as jnp
from jax import lax
from jax.experimental import pallas as pl
from jax.experimental.pallas import tpu as pltpu
from jax.experimental.pallas import tpu_sc as plsc

_CLASS_RANGE = (40., 40., 40., 50., 50., 50., 50., 50., 50., 30., 30.)
_NCLS = 11
_IOU_THR = 0.3
_B = 256
_NP = 7936
_NB = _NP // _B

_SC_NC = 2
_SC_NS = 16
_NW = _SC_NC * _SC_NS
_PACKD = 128
_VPAD = 5120


def _sc_permute(packed, src):
    b_per_w = _NP // _NW
    chunk = 248
    mesh = plsc.VectorSubcoreMesh(core_axis_name="c", subcore_axis_name="s")

    @functools.partial(
        pl.kernel, mesh=mesh,
        out_type=jax.ShapeDtypeStruct((_NP, _PACKD), jnp.float32),
        scratch_types=[
            pltpu.VMEM((chunk,), jnp.int32),
            pltpu.VMEM((chunk, _PACKD), jnp.float32),
            pltpu.SemaphoreType.DMA,
        ],
    )
    def k(packed_hbm, src_hbm, out_hbm, idx_v, rows_v, sem):
        wid = lax.axis_index("s") * _SC_NC + lax.axis_index("c")
        for j in range(b_per_w // chunk):
            base = wid * b_per_w + j * chunk
            pltpu.sync_copy(src_hbm.at[pl.ds(base, chunk)], idx_v)
            pltpu.async_copy(packed_hbm.at[idx_v], rows_v, sem).wait()
            pltpu.sync_copy(rows_v, out_hbm.at[pl.ds(base, chunk)])

    return k(packed, src)


def _prep_kernel(b3t_ref, s_ref, s3_ref, cls_ref, ext_ref,
                 scores_ref, feat_ref, validf_ref, b3o_ref):
    cx = b3t_ref[0:1, :]
    cy = b3t_ref[1:2, :]
    cz = b3t_ref[2:3, :]
    clsf = cls_ref[0:1, :].astype(jnp.float32)
    rng = jnp.zeros_like(cx)
    for k, r in enumerate(_CLASS_RANGE):
        rng = jnp.where(clsf == float(k), r, rng)
    dist = jnp.sqrt(cx * cx + cy * cy + cz * cz)
    validf = (dist < rng).astype(jnp.float32)
    validf_ref[0:1, :] = validf
    sc = s_ref[0:1, :] * s3_ref[0:1, :] * validf
    scores_ref[0:1, :] = sc

    def _bf(v):
        return v.astype(jnp.bfloat16).astype(jnp.float32)

    R = [[ext_ref[i, j] for j in range(3)] for i in range(3)]
    t = [ext_ref[i, 3] for i in range(3)]
    cxb, cyb, czb = _bf(cx), _bf(cy), _bf(cz)
    cw = [cxb * R[i][0] + cyb * R[i][1] + czb * R[i][2] + t[i]
          for i in range(3)]
    for i in range(3):
        b3o_ref[i:i + 1, :] = cw[i]
        b3o_ref[3 + i:4 + i, :] = b3t_ref[3 + i:4 + i, :]
    o6 = _bf(b3t_ref[6:7, :])
    o7 = _bf(b3t_ref[7:8, :])
    o8 = _bf(b3t_ref[8:9, :])
    v9 = _bf(b3t_ref[9:10, :])
    v10 = _bf(b3t_ref[10:11, :])
    v11 = _bf(b3t_ref[11:12, :])
    for i in range(3):
        b3o_ref[6 + i:7 + i, :] = o6 * R[i][0] + o7 * R[i][1] + o8 * R[i][2]
        b3o_ref[9 + i:10 + i, :] = v9 * R[i][0] + v10 * R[i][1] + v11 * R[i][2]

    w = jnp.abs(b3t_ref[3:4, :]) + 0.5
    l = jnp.abs(b3t_ref[5:6, :]) + 0.5
    x = cw[0]
    z = cw[2]
    feat_ref[0:1, :] = x - w * 0.5
    feat_ref[1:2, :] = x + w * 0.5
    feat_ref[2:3, :] = z - l * 0.5
    feat_ref[3:4, :] = z + l * 0.5
    feat_ref[4:5, :] = w * l


def _nms_kernel(start_ref, rowref, colref, keep_ref, krow_buf):
    b = pl.program_id(0)

    @pl.when(b == 0)
    def _init():
        keep_ref[:, :] = jnp.zeros((_NP, 1), jnp.float32)
        krow_buf[:, :] = jnp.zeros((1, _NP), jnp.float32)

    blk = pl.ds(b * _B, _B)
    bx1 = rowref[0:1, blk]
    bx2 = rowref[1:2, blk]
    bz1 = rowref[2:3, blk]
    bz2 = rowref[3:4, blk]
    bar = rowref[4:5, blk]
    bcls = rowref[5:6, blk]
    cx1 = colref[blk, 0:1]
    cx2 = colref[blk, 1:2]
    cz1 = colref[blk, 2:3]
    cz2 = colref[blk, 3:4]
    car = colref[blk, 4:5]
    ccls = colref[blk, 5:6]

    def _sup_rowform(ch):
        ax1 = colref[ch, 0:1]
        ax2 = colref[ch, 1:2]
        az1 = colref[ch, 2:3]
        az2 = colref[ch, 3:4]
        aar = colref[ch, 4:5]
        acls = colref[ch, 5:6]
        ix = jnp.maximum(jnp.minimum(ax2, bx2) - jnp.maximum(ax1, bx1), 0.0)
        iz = jnp.maximum(jnp.minimum(az2, bz2) - jnp.maximum(az1, bz1), 0.0)
        inter = ix * iz
        union = aar + bar - inter
        iou = inter / jnp.maximum(union, 1e-9)
        return jnp.logical_and(iou > _IOU_THR, acls == bcls).astype(jnp.float32)

    def _sup_colform(ch):
        ax1 = rowref[0:1, ch]
        ax2 = rowref[1:2, ch]
        az1 = rowref[2:3, ch]
        az2 = rowref[3:4, ch]
        aar = rowref[4:5, ch]
        acls = rowref[5:6, ch]
        ix = jnp.maximum(jnp.minimum(ax2, cx2) - jnp.maximum(ax1, cx1), 0.0)
        iz = jnp.maximum(jnp.minimum(az2, cz2) - jnp.maximum(az1, cz1), 0.0)
        inter = ix * iz
        union = aar + car - inter
        iou = inter / jnp.maximum(union, 1e-9)
        return jnp.logical_and(iou > _IOU_THR, acls == ccls).astype(jnp.float32)

    def chunk_body(kb, pre):
        ch = pl.ds(kb * _B, _B)
        kcol = keep_ref[ch, 0:1]
        krow = krow_buf[0:1, ch]
        pre_row = jnp.maximum(
            pre[0], jnp.max(_sup_rowform(ch) * kcol, axis=0, keepdims=True))
        pre_col = jnp.maximum(
            pre[1], jnp.max(_sup_colform(ch) * krow, axis=1, keepdims=True))
        return (pre_row, pre_col)

    pre_row, pre_col = jax.lax.fori_loop(
        start_ref[b], b, chunk_body,
        (jnp.zeros((1, _B), jnp.float32), jnp.zeros((_B, 1), jnp.float32)))

    sub = jax.lax.broadcasted_iota(jnp.int32, (_B, _B), 0)
    lan = jax.lax.broadcasted_iota(jnp.int32, (_B, _B), 1)
    m = _sup_rowform(blk)
    mlow = m * (sub < lan).astype(jnp.float32)
    mupp = m * (lan < sub).astype(jnp.float32)

    def _row_phase(alive_col):
        dead_row = jnp.maximum(
            pre_row, jnp.max(mlow * alive_col, axis=0, keepdims=True))
        return 1.0 - dead_row

    def cond(c):
        return c[1] > 0.0

    def body(c):
        alive_col, _ = c
        alive_row = _row_phase(alive_col)
        dead_col = jnp.maximum(
            pre_col, jnp.max(mupp * alive_row, axis=1, keepdims=True))
        new_col = 1.0 - dead_col
        changed = jnp.max(jnp.abs(new_col - alive_col))
        return (new_col, changed)

    alive_col, _ = jax.lax.while_loop(
        cond, body, (jnp.ones((_B, 1), jnp.float32), jnp.float32(1.0)))
    keep_ref[blk, 0:1] = alive_col
    krow_buf[0:1, blk] = _row_phase(alive_col)


def _out_kernel(boxes_ref, b3_ref, sc_ref, emb_ref, keep_ref, out_ref):
    k = keep_ref[:, 0:1]
    out_ref[:, 0:4] = boxes_ref[:, :] * k
    out_ref[:, 4:16] = b3_ref[:, :] * k
    out_ref[:, 16:17] = sc_ref[:, :] * k
    out_ref[:, 17:273] = emb_ref[:, :] * k


def _run_prep(b3t, s, s3, cls, extrinsics):
    n = s.shape[1]
    rot = extrinsics[:3, :3].astype(jnp.bfloat16).astype(jnp.float32)
    extrinsics = jnp.concatenate(
        [jnp.concatenate([rot, extrinsics[:3, 3:4]], axis=1),
         extrinsics[3:4, :]], axis=0)
    return pl.pallas_call(
        _prep_kernel,
        in_specs=[
            pl.BlockSpec(memory_space=pltpu.VMEM),
            pl.BlockSpec(memory_space=pltpu.VMEM),
            pl.BlockSpec(memory_space=pltpu.VMEM),
            pl.BlockSpec(memory_space=pltpu.VMEM),
            pl.BlockSpec(memory_space=pltpu.SMEM),
        ],
        out_shape=(
            jax.ShapeDtypeStruct((1, n), jnp.float32),
            jax.ShapeDtypeStruct((5, n), jnp.float32),
            jax.ShapeDtypeStruct((1, n), jnp.float32),
            jax.ShapeDtypeStruct((12, n), jnp.float32),
        ),
    )(b3t, s, s3, cls, extrinsics)


def _run_nms(start_blk, perm):
    perm = perm[:, :8]
    permT = perm.T
    return pl.pallas_call(
        _nms_kernel,
        grid=(_NB,),
        in_specs=[
            pl.BlockSpec(memory_space=pltpu.SMEM),
            pl.BlockSpec(permT.shape, lambda b: (0, 0)),
            pl.BlockSpec(perm.shape, lambda b: (0, 0)),
        ],
        out_specs=pl.BlockSpec((_NP, 1), lambda b: (0, 0)),
        out_shape=jax.ShapeDtypeStruct((_NP, 1), jnp.float32),
        scratch_shapes=[pltpu.VMEM((1, _NP), jnp.float32)],
        compiler_params=pltpu.CompilerParams(
            dimension_semantics=("arbitrary",)),
    )(start_blk, permT, perm)


def kernel(det_boxes, det_scores, det_boxes_3d, det_scores_3d, embeddings,
           extrinsics, det_class_ids):
    n = det_scores.shape[0]
    b3t = det_boxes_3d.T
    s = det_scores.reshape(1, n)
    s3 = det_scores_3d.reshape(1, n)
    cls = det_class_ids.astype(jnp.int32).reshape(1, n)

    scores, feat, validf, b3o = _run_prep(b3t, s, s3, cls, extrinsics)

    scores1 = scores[0]
    clsi = cls[0]
    key = clsi.astype(jnp.float32) * 2.0 - scores1
    order = jnp.argsort(key)
    cs = clsi[order]
    counts = jnp.bincount(clsi, length=_NCLS)
    padded = ((counts + _B - 1) // _B) * _B
    seg_off = jnp.concatenate(
        [jnp.zeros((1,), counts.dtype), jnp.cumsum(padded)[:-1]])
    cls_start = jnp.concatenate(
        [jnp.zeros((1,), counts.dtype), jnp.cumsum(counts)[:-1]])
    pos = seg_off[cs] + (jnp.arange(n, dtype=counts.dtype) - cls_start[cs])
    packT = jnp.zeros((_PACKD, _VPAD), jnp.float32)
    packT = packT.at[0:5, :n].set(feat)
    packT = packT.at[5, :].set(-1.0).at[5, :n].set(clsi.astype(jnp.float32))
    packed = packT.T
    src = jnp.full((_NP,), n, jnp.int32).at[pos].set(order.astype(jnp.int32))
    perm = _sc_permute(packed, src)
    bidx = jnp.arange(_NB, dtype=jnp.int32)
    seg_end = jnp.cumsum(padded)
    cob = jnp.searchsorted(seg_end, bidx * _B, side="right")
    total = seg_end[-1]
    start_blk = jnp.where(
        bidx * _B < total,
        (seg_off[jnp.minimum(cob, _NCLS - 1)] // _B).astype(jnp.int32),
        bidx)
    posoforig = jnp.zeros((n,), jnp.int32).at[order].set(pos.astype(jnp.int32))
    keep = _run_nms(start_blk, perm)
    keepf = (keep[posoforig, 0] * validf[0]).reshape(n, 1)

    out = pl.pallas_call(
        _out_kernel,
        out_shape=jax.ShapeDtypeStruct((n, 273), jnp.float32),
    )(det_boxes, b3o.T, scores1.reshape(n, 1), embeddings, keepf)
    return out


# speedup vs baseline: 1.2794x; 1.2794x over previous
import functools

import jax
import jax.numpy as jnp
from jax import lax
from jax.experimental import pallas as pl
from jax.experimental.pallas import tpu as pltpu
from jax.experimental.pallas import tpu_sc as plsc

_CLASS_RANGE = (40., 40., 40., 50., 50., 50., 50., 50., 50., 30., 30.)
_NCLS = 11
_IOU_THR = 0.3
_B = 128
_NP = 6400
_NB = _NP // _B

_SC_NC = 2
_SC_NS = 16
_NW = _SC_NC * _SC_NS
_PACKD = 128
_VPAD = 5008


def _sc_permute(packed, src):
    b_per_w = _NP // _NW
    chunk = 200
    mesh = plsc.VectorSubcoreMesh(core_axis_name="c", subcore_axis_name="s")

    @functools.partial(
        pl.kernel, mesh=mesh,
        out_type=jax.ShapeDtypeStruct((_NP, _PACKD), jnp.float32),
        scratch_types=[
            pltpu.VMEM((chunk,), jnp.int32),
            pltpu.VMEM((chunk, _PACKD), jnp.float32),
            pltpu.SemaphoreType.DMA,
        ],
    )
    def k(packed_hbm, src_hbm, out_hbm, idx_v, rows_v, sem):
        wid = lax.axis_index("s") * _SC_NC + lax.axis_index("c")
        for j in range(b_per_w // chunk):
            base = wid * b_per_w + j * chunk
            pltpu.sync_copy(src_hbm.at[pl.ds(base, chunk)], idx_v)
            pltpu.async_copy(packed_hbm.at[idx_v], rows_v, sem).wait()
            pltpu.sync_copy(rows_v, out_hbm.at[pl.ds(base, chunk)])

    return k(packed, src)


def _prep_kernel(b3t_ref, s_ref, s3_ref, cls_ref, ext_ref,
                 scores_ref, feat_ref, validf_ref, b3o_ref):
    cx = b3t_ref[0:1, :]
    cy = b3t_ref[1:2, :]
    cz = b3t_ref[2:3, :]
    clsf = cls_ref[0:1, :].astype(jnp.float32)
    rng = jnp.zeros_like(cx)
    for k, r in enumerate(_CLASS_RANGE):
        rng = jnp.where(clsf == float(k), r, rng)
    dist = jnp.sqrt(cx * cx + cy * cy + cz * cz)
    validf = (dist < rng).astype(jnp.float32)
    validf_ref[0:1, :] = validf
    sc = s_ref[0:1, :] * s3_ref[0:1, :] * validf
    scores_ref[0:1, :] = sc

    def _bf(v):
        return v.astype(jnp.bfloat16).astype(jnp.float32)

    R = [[ext_ref[i, j] for j in range(3)] for i in range(3)]
    t = [ext_ref[i, 3] for i in range(3)]
    cxb, cyb, czb = _bf(cx), _bf(cy), _bf(cz)
    cw = [cxb * R[i][0] + cyb * R[i][1] + czb * R[i][2] + t[i]
          for i in range(3)]
    for i in range(3):
        b3o_ref[i:i + 1, :] = cw[i]
        b3o_ref[3 + i:4 + i, :] = b3t_ref[3 + i:4 + i, :]
    o6 = _bf(b3t_ref[6:7, :])
    o7 = _bf(b3t_ref[7:8, :])
    o8 = _bf(b3t_ref[8:9, :])
    v9 = _bf(b3t_ref[9:10, :])
    v10 = _bf(b3t_ref[10:11, :])
    v11 = _bf(b3t_ref[11:12, :])
    for i in range(3):
        b3o_ref[6 + i:7 + i, :] = o6 * R[i][0] + o7 * R[i][1] + o8 * R[i][2]
        b3o_ref[9 + i:10 + i, :] = v9 * R[i][0] + v10 * R[i][1] + v11 * R[i][2]

    w = jnp.abs(b3t_ref[3:4, :]) + 0.5
    l = jnp.abs(b3t_ref[5:6, :]) + 0.5
    x = cw[0]
    z = cw[2]
    feat_ref[0:1, :] = x - w * 0.5
    feat_ref[1:2, :] = x + w * 0.5
    feat_ref[2:3, :] = z - l * 0.5
    feat_ref[3:4, :] = z + l * 0.5
    feat_ref[4:5, :] = w * l


def _nms_kernel(start_ref, rowref, colref, keep_ref, krow_buf):
    b = pl.program_id(0)

    @pl.when(b == 0)
    def _init():
        keep_ref[:, :] = jnp.zeros((_NP, 1), jnp.float32)
        krow_buf[:, :] = jnp.zeros((1, _NP), jnp.float32)

    blk = pl.ds(b * _B, _B)
    bx1 = rowref[0:1, blk]
    bx2 = rowref[1:2, blk]
    bz1 = rowref[2:3, blk]
    bz2 = rowref[3:4, blk]
    bar = rowref[4:5, blk]
    bcls = rowref[5:6, blk]
    cx1 = colref[blk, 0:1]
    cx2 = colref[blk, 1:2]
    cz1 = colref[blk, 2:3]
    cz2 = colref[blk, 3:4]
    car = colref[blk, 4:5]
    ccls = colref[blk, 5:6]

    def _sup_rowform(ch):
        ax1 = colref[ch, 0:1]
        ax2 = colref[ch, 1:2]
        az1 = colref[ch, 2:3]
        az2 = colref[ch, 3:4]
        aar = colref[ch, 4:5]
        acls = colref[ch, 5:6]
        ix = jnp.maximum(jnp.minimum(ax2, bx2) - jnp.maximum(ax1, bx1), 0.0)
        iz = jnp.maximum(jnp.minimum(az2, bz2) - jnp.maximum(az1, bz1), 0.0)
        inter = ix * iz
        union = aar + bar - inter
        iou = inter / jnp.maximum(union, 1e-9)
        return jnp.logical_and(iou > _IOU_THR, acls == bcls).astype(jnp.float32)

    def _sup_colform(ch):
        ax1 = rowref[0:1, ch]
        ax2 = rowref[1:2, ch]
        az1 = rowref[2:3, ch]
        az2 = rowref[3:4, ch]
        aar = rowref[4:5, ch]
        acls = rowref[5:6, ch]
        ix = jnp.maximum(jnp.minimum(ax2, cx2) - jnp.maximum(ax1, cx1), 0.0)
        iz = jnp.maximum(jnp.minimum(az2, cz2) - jnp.maximum(az1, cz1), 0.0)
        inter = ix * iz
        union = aar + car - inter
        iou = inter / jnp.maximum(union, 1e-9)
        return jnp.logical_and(iou > _IOU_THR, acls == ccls).astype(jnp.float32)

    def chunk_body(kb, pre):
        ch = pl.ds(kb * _B, _B)
        kcol = keep_ref[ch, 0:1]
        krow = krow_buf[0:1, ch]
        pre_row = jnp.maximum(
            pre[0], jnp.max(_sup_rowform(ch) * kcol, axis=0, keepdims=True))
        pre_col = jnp.maximum(
            pre[1], jnp.max(_sup_colform(ch) * krow, axis=1, keepdims=True))
        return (pre_row, pre_col)

    pre_row, pre_col = jax.lax.fori_loop(
        start_ref[b], b, chunk_body,
        (jnp.zeros((1, _B), jnp.float32), jnp.zeros((_B, 1), jnp.float32)))

    sub = jax.lax.broadcasted_iota(jnp.int32, (_B, _B), 0)
    lan = jax.lax.broadcasted_iota(jnp.int32, (_B, _B), 1)
    m = _sup_rowform(blk)
    mlow = m * (sub < lan).astype(jnp.float32)
    mupp = m * (lan < sub).astype(jnp.float32)

    def _row_phase(alive_col):
        dead_row = jnp.maximum(
            pre_row, jnp.max(mlow * alive_col, axis=0, keepdims=True))
        return 1.0 - dead_row

    def cond(c):
        return c[1] > 0.0

    def body(c):
        alive_col, _ = c
        alive_row = _row_phase(alive_col)
        dead_col = jnp.maximum(
            pre_col, jnp.max(mupp * alive_row, axis=1, keepdims=True))
        new_col = 1.0 - dead_col
        changed = jnp.max(jnp.abs(new_col - alive_col))
        return (new_col, changed)

    alive_col, _ = jax.lax.while_loop(
        cond, body, (jnp.ones((_B, 1), jnp.float32), jnp.float32(1.0)))
    keep_ref[blk, 0:1] = alive_col
    krow_buf[0:1, blk] = _row_phase(alive_col)


def _out_kernel(boxes_ref, b3_ref, sc_ref, emb_ref, keep_ref, out_ref):
    k = keep_ref[:, 0:1]
    out_ref[:, 0:4] = boxes_ref[:, :] * k
    out_ref[:, 4:16] = b3_ref[:, :] * k
    out_ref[:, 16:17] = sc_ref[:, :] * k
    out_ref[:, 17:273] = emb_ref[:, :] * k


def _run_prep(b3t, s, s3, cls, extrinsics):
    n = s.shape[1]
    rot = extrinsics[:3, :3].astype(jnp.bfloat16).astype(jnp.float32)
    extrinsics = jnp.concatenate(
        [jnp.concatenate([rot, extrinsics[:3, 3:4]], axis=1),
         extrinsics[3:4, :]], axis=0)
    return pl.pallas_call(
        _prep_kernel,
        in_specs=[
            pl.BlockSpec(memory_space=pltpu.VMEM),
            pl.BlockSpec(memory_space=pltpu.VMEM),
            pl.BlockSpec(memory_space=pltpu.VMEM),
            pl.BlockSpec(memory_space=pltpu.VMEM),
            pl.BlockSpec(memory_space=pltpu.SMEM),
        ],
        out_shape=(
            jax.ShapeDtypeStruct((1, n), jnp.float32),
            jax.ShapeDtypeStruct((5, n), jnp.float32),
            jax.ShapeDtypeStruct((1, n), jnp.float32),
            jax.ShapeDtypeStruct((12, n), jnp.float32),
        ),
    )(b3t, s, s3, cls, extrinsics)


def _run_nms(start_blk, perm):
    perm = perm[:, :8]
    permT = perm.T
    return pl.pallas_call(
        _nms_kernel,
        grid=(_NB,),
        in_specs=[
            pl.BlockSpec(memory_space=pltpu.SMEM),
            pl.BlockSpec(permT.shape, lambda b: (0, 0)),
            pl.BlockSpec(perm.shape, lambda b: (0, 0)),
        ],
        out_specs=pl.BlockSpec((_NP, 1), lambda b: (0, 0)),
        out_shape=jax.ShapeDtypeStruct((_NP, 1), jnp.float32),
        scratch_shapes=[pltpu.VMEM((1, _NP), jnp.float32)],
        compiler_params=pltpu.CompilerParams(
            dimension_semantics=("arbitrary",)),
    )(start_blk, permT, perm)


def kernel(det_boxes, det_scores, det_boxes_3d, det_scores_3d, embeddings,
           extrinsics, det_class_ids):
    n = det_scores.shape[0]
    b3t = det_boxes_3d.T
    s = det_scores.reshape(1, n)
    s3 = det_scores_3d.reshape(1, n)
    cls = det_class_ids.astype(jnp.int32).reshape(1, n)

    scores, feat, validf, b3o = _run_prep(b3t, s, s3, cls, extrinsics)

    scores1 = scores[0]
    clsi = cls[0]
    key = clsi.astype(jnp.float32) * 2.0 - scores1
    order = jnp.argsort(key)
    cs = clsi[order]
    counts = jnp.sum(
        (clsi[None, :] == jnp.arange(_NCLS, dtype=clsi.dtype)[:, None])
        .astype(jnp.int32), axis=1)
    padded = ((counts + _B - 1) // _B) * _B
    seg_off = jnp.concatenate(
        [jnp.zeros((1,), counts.dtype), jnp.cumsum(padded)[:-1]])
    cls_start = jnp.concatenate(
        [jnp.zeros((1,), counts.dtype), jnp.cumsum(counts)[:-1]])
    pos = seg_off[cs] + (jnp.arange(n, dtype=counts.dtype) - cls_start[cs])
    packT = jnp.zeros((_PACKD, _VPAD), jnp.float32)
    packT = packT.at[0:5, :n].set(feat)
    packT = packT.at[5, :].set(-1.0).at[5, :n].set(clsi.astype(jnp.float32))
    packed = packT.T
    src = jnp.full((_NP,), n, jnp.int32).at[pos].set(order.astype(jnp.int32))
    perm = _sc_permute(packed, src)
    bidx = jnp.arange(_NB, dtype=jnp.int32)
    seg_end = jnp.cumsum(padded)
    cob = jnp.sum((seg_end[:, None] <= (bidx * _B)[None, :]).astype(jnp.int32),
                  axis=0)
    total = seg_end[-1]
    start_blk = jnp.where(
        bidx * _B < total,
        (seg_off[jnp.minimum(cob, _NCLS - 1)] // _B).astype(jnp.int32),
        bidx)
    posoforig = jnp.zeros((n,), jnp.int32).at[order].set(pos.astype(jnp.int32))
    keep = _run_nms(start_blk, perm)
    keepf = (keep[posoforig, 0] * validf[0]).reshape(n, 1)

    out = pl.pallas_call(
        _out_kernel,
        out_shape=jax.ShapeDtypeStruct((n, 273), jnp.float32),
    )(det_boxes, b3o.T, scores1.reshape(n, 1), embeddings, keepf)
    return out
